# baseline (speedup 1.0000x reference)
import functools

import jax
import jax.numpy as jnp
from jax import lax
from jax.experimental import pallas as pl
from jax.experimental.pallas import tpu as pltpu
from jax.experimental.pallas import tpu_sc as plsc

NUM_FIELDS = 26
VOCAB = 100000
D = 32
B = 16384
N = B * NUM_FIELDS
NC, NS, L = 2, 16, 16
NW = NC * NS
BPW = N // NW
CHUNK = 1664
NCHUNK = BPW // CHUNK
GSIZE = 128
NG = CHUNK // GSIZE


@functools.cache
def _make_sc_gather():
    mesh = plsc.VectorSubcoreMesh(core_axis_name="c", subcore_axis_name="s")

    @functools.partial(
        pl.kernel,
        out_type=jax.ShapeDtypeStruct((N, D), jnp.float32),
        mesh=mesh,
        compiler_params=pltpu.CompilerParams(use_tc_tiling_on_sc=False),
        scratch_types=[
            pltpu.VMEM((CHUNK,), jnp.int32),
            pltpu.VMEM((CHUNK,), jnp.int32),
            pltpu.VMEM((CHUNK, D), jnp.float32),
            pltpu.SemaphoreType.DMA,
        ],
    )
    def k(idx_hbm, tab_hbm, offs_hbm, out_hbm, offs_v, idx_v, rows_v, sem):
        wid = lax.axis_index("s") * NC + lax.axis_index("c")
        base = wid * BPW
        pltpu.sync_copy(offs_hbm, offs_v)

        def chunk_body(c, carry):
            row0 = base + c * CHUNK
            pltpu.sync_copy(idx_hbm.at[pl.ds(row0, CHUNK)], idx_v)

            def add_body(i, carry2):
                s16 = pl.ds(i * L, L)
                idx_v[s16] = idx_v[s16] + offs_v[s16]
                return carry2

            lax.fori_loop(0, CHUNK // L, add_body, 0)

            cps = []
            for j in range(NG):
                sl = pl.ds(j * GSIZE, GSIZE)
                cps.append(
                    pltpu.async_copy(tab_hbm.at[idx_v.at[sl]], rows_v.at[sl], sem)
                )
            for cp in cps:
                cp.wait()
            pltpu.sync_copy(rows_v, out_hbm.at[pl.ds(row0, CHUNK)])
            return carry

        lax.fori_loop(0, NCHUNK, chunk_body, 0)

    return k


def kernel(sparse_features, tables):
    idx_flat = sparse_features.reshape(-1).astype(jnp.int32)
    offs = jnp.tile(jnp.arange(NUM_FIELDS, dtype=jnp.int32) * VOCAB, CHUNK // NUM_FIELDS)
    out = _make_sc_gather()(idx_flat, tables, offs)
    return out.reshape(B, NUM_FIELDS, D)


# speedup vs baseline: 1.0856x; 1.0856x over previous
import functools

import jax
import jax.numpy as jnp
from jax import lax
from jax.experimental import pallas as pl
from jax.experimental.pallas import tpu as pltpu
from jax.experimental.pallas import tpu_sc as plsc

NUM_FIELDS = 26
VOCAB = 100000
D = 32
B = 16384
NC, NS, L = 2, 16, 16
NW = NC * NS
BPW = B // NW
GSIZE = 128
NG = BPW // GSIZE


@functools.cache
def _make_sc_gather():
    mesh = plsc.VectorSubcoreMesh(core_axis_name="c", subcore_axis_name="s")

    @functools.partial(
        pl.kernel,
        out_type=jax.ShapeDtypeStruct((B, NUM_FIELDS * D), jnp.float32),
        mesh=mesh,
        compiler_params=pltpu.CompilerParams(use_tc_tiling_on_sc=False),
        scratch_types=[
            pltpu.VMEM((BPW,), jnp.int32),
            pltpu.VMEM((BPW, D), jnp.float32),
            pltpu.SemaphoreType.DMA,
        ],
    )
    def k(sfT_hbm, tab_hbm, out_hbm, idx_v, rows_v, sem):
        wid = lax.axis_index("s") * NC + lax.axis_index("c")
        b0 = wid * BPW

        def field_body(f, carry):
            pltpu.sync_copy(sfT_hbm.at[f, pl.ds(b0, BPW)], idx_v)

            def add_body(i, carry2):
                s16 = pl.ds(i * L, L)
                idx_v[s16] = idx_v[s16] + f * VOCAB
                return carry2

            lax.fori_loop(0, BPW // L, add_body, 0)

            cps = []
            for j in range(NG):
                sl = pl.ds(j * GSIZE, GSIZE)
                cps.append(
                    pltpu.async_copy(tab_hbm.at[idx_v.at[sl]], rows_v.at[sl], sem)
                )
            for cp in cps:
                cp.wait()
            pltpu.sync_copy(rows_v, out_hbm.at[pl.ds(b0, BPW), pl.ds(f * D, D)])
            return carry

        lax.fori_loop(0, NUM_FIELDS, field_body, 0)

    return k


def kernel(sparse_features, tables):
    lin = jnp.reshape(tables, (NUM_FIELDS * VOCAB * D // 128, 128))
    lin = lax.optimization_barrier(lin)
    tab_rm = jnp.reshape(lin, (NUM_FIELDS * VOCAB, D))
    sfT = sparse_features.T.astype(jnp.int32)
    out = _make_sc_gather()(sfT, tab_rm)
    return out.reshape(B, NUM_FIELDS, D)


# speedup vs baseline: 1.2179x; 1.1219x over previous
import functools

import jax
import jax.numpy as jnp
from jax import lax
from jax.experimental import pallas as pl
from jax.experimental.pallas import tpu as pltpu
from jax.experimental.pallas import tpu_sc as plsc

NUM_FIELDS = 26
VOCAB = 100000
D = 32
B = 16384
NC, NS, L = 2, 16, 16
NW = NC * NS
BPW = B // NW
GSIZE = 128
NG = BPW // GSIZE

ROWS = NUM_FIELDS * VOCAB
BV = 2048
NBLK = -(-ROWS // BV)


def _tr_body(in_ref, out_ref, s_ref):
    s_ref[...] = in_ref[...].T
    for c in range(4):
        out_ref[:, c * D:(c + 1) * D] = s_ref[pl.Slice(c, BV // 4, 4), :]


@functools.cache
def _make_tc_transpose():
    return pl.pallas_call(
        _tr_body,
        grid=(NBLK,),
        in_specs=[pl.BlockSpec((D, BV), lambda i: (0, i))],
        out_specs=pl.BlockSpec((BV // 4, 4 * D), lambda i: (i, 0)),
        out_shape=jax.ShapeDtypeStruct((ROWS // 4, 4 * D), jnp.float32),
        scratch_shapes=[pltpu.VMEM((BV, D), jnp.float32)],
        compiler_params=pltpu.CompilerParams(
            dimension_semantics=("arbitrary",),
        ),
    )


@functools.cache
def _make_sc_gather():
    mesh = plsc.VectorSubcoreMesh(core_axis_name="c", subcore_axis_name="s")

    @functools.partial(
        pl.kernel,
        out_type=jax.ShapeDtypeStruct((B, NUM_FIELDS * D), jnp.float32),
        mesh=mesh,
        compiler_params=pltpu.CompilerParams(use_tc_tiling_on_sc=False),
        scratch_types=[
            pltpu.VMEM((BPW,), jnp.int32),
            pltpu.VMEM((BPW, D), jnp.float32),
            pltpu.SemaphoreType.DMA,
        ],
    )
    def k(sfT_hbm, tab_hbm, out_hbm, idx_v, rows_v, sem):
        wid = lax.axis_index("s") * NC + lax.axis_index("c")
        b0 = wid * BPW

        def field_body(f, carry):
            pltpu.sync_copy(sfT_hbm.at[f, pl.ds(b0, BPW)], idx_v)

            def add_body(i, carry2):
                s16 = pl.ds(i * L, L)
                idx_v[s16] = idx_v[s16] + f * VOCAB
                return carry2

            lax.fori_loop(0, BPW // L, add_body, 0)

            cps = []
            for j in range(NG):
                sl = pl.ds(j * GSIZE, GSIZE)
                cps.append(
                    pltpu.async_copy(tab_hbm.at[idx_v.at[sl]], rows_v.at[sl], sem)
                )
            for cp in cps:
                cp.wait()
            pltpu.sync_copy(rows_v, out_hbm.at[pl.ds(b0, BPW), pl.ds(f * D, D)])
            return carry

        lax.fori_loop(0, NUM_FIELDS, field_body, 0)

    return k


def kernel(sparse_features, tables):
    tabT = tables.T
    lin = _make_tc_transpose()(tabT)
    tab_rm = jnp.reshape(lin, (ROWS, D))
    sfT = sparse_features.T.astype(jnp.int32)
    out = _make_sc_gather()(sfT, tab_rm)
    return out.reshape(B, NUM_FIELDS, D)


# speedup vs baseline: 1.5465x; 1.2698x over previous
import functools

import jax
import jax.numpy as jnp
from jax import lax
from jax.experimental import pallas as pl
from jax.experimental.pallas import tpu as pltpu
from jax.experimental.pallas import tpu_sc as plsc

NUM_FIELDS = 26
VOCAB = 100000
D = 32
B = 16384
NC, NS, L = 2, 16, 16
NW = NC * NS
BPW = B // NW
GSIZE = 128
NG = BPW // GSIZE

ROWS = NUM_FIELDS * VOCAB
SB = 512
NSB = -(-ROWS // SB)
K = 4
BV = K * SB
NBLK = -(-NSB // K)
LINES = NSB * 128
TROWS = LINES * 4


def _tr_body(in_ref, out_ref):
    for g in range(K):
        x = jnp.concatenate(
            [in_ref[:, g * SB + 128 * c:g * SB + 128 * (c + 1)] for c in range(4)],
            axis=0,
        )
        out_ref[g * 128:(g + 1) * 128, :] = x.T


@functools.cache
def _make_tc_transpose():
    return pl.pallas_call(
        _tr_body,
        grid=(NBLK,),
        in_specs=[pl.BlockSpec((D, BV), lambda i: (0, i))],
        out_specs=pl.BlockSpec((K * 128, 128), lambda i: (i, 0)),
        out_shape=jax.ShapeDtypeStruct((NBLK * K * 128, 128), jnp.float32),
        compiler_params=pltpu.CompilerParams(
            dimension_semantics=("arbitrary",),
        ),
    )


@functools.cache
def _make_sc_gather():
    mesh = plsc.VectorSubcoreMesh(core_axis_name="c", subcore_axis_name="s")

    @functools.partial(
        pl.kernel,
        out_type=jax.ShapeDtypeStruct((B, NUM_FIELDS * D), jnp.float32),
        mesh=mesh,
        compiler_params=pltpu.CompilerParams(use_tc_tiling_on_sc=False),
        scratch_types=[
            pltpu.VMEM((BPW,), jnp.int32),
            pltpu.VMEM((BPW, D), jnp.float32),
            pltpu.SemaphoreType.DMA,
        ],
    )
    def k(sfT_hbm, tab_hbm, out_hbm, idx_v, rows_v, sem):
        wid = lax.axis_index("s") * NC + lax.axis_index("c")
        b0 = wid * BPW

        def field_body(f, carry):
            pltpu.sync_copy(sfT_hbm.at[f, pl.ds(b0, BPW)], idx_v)

            def remap_body(i, carry2):
                s16 = pl.ds(i * L, L)
                v = idx_v[s16] + f * VOCAB
                idx_v[s16] = (
                    ((v >> 9) << 9) + ((v & 127) << 2) + ((v >> 7) & 3)
                )
                return carry2

            lax.fori_loop(0, BPW // L, remap_body, 0)

            cps = []
            for j in range(NG):
                sl = pl.ds(j * GSIZE, GSIZE)
                cps.append(
                    pltpu.async_copy(tab_hbm.at[idx_v.at[sl]], rows_v.at[sl], sem)
                )
            for cp in cps:
                cp.wait()
            pltpu.sync_copy(rows_v, out_hbm.at[pl.ds(b0, BPW), pl.ds(f * D, D)])
            return carry

        lax.fori_loop(0, NUM_FIELDS, field_body, 0)

    return k


def kernel(sparse_features, tables):
    tabT = tables.T
    lin = _make_tc_transpose()(tabT)
    tab_rm = jnp.reshape(lin, (lin.shape[0] * 4, D))
    sfT = sparse_features.T.astype(jnp.int32)
    out = _make_sc_gather()(sfT, tab_rm)
    return out.reshape(B, NUM_FIELDS, D)


# speedup vs baseline: 2.8864x; 1.8664x over previous
import functools

import jax
import jax.numpy as jnp
from jax import lax
from jax.experimental import pallas as pl
from jax.experimental.pallas import tpu as pltpu
from jax.experimental.pallas import tpu_sc as plsc

NUM_FIELDS = 26
VOCAB = 100000
D = 32
B = 16384
NC, NS, L = 2, 16, 16
NW = NC * NS
BPW = B // NW
GSIZE = 128
NG = BPW // GSIZE

ROWS = NUM_FIELDS * VOCAB
SB = 512
NSB = -(-ROWS // SB)
K = 16
BV = K * SB
NBLK = -(-NSB // K)
LINES = NSB * 128
TROWS = LINES * 4


def _tr_body(in_ref, out_ref):
    for g in range(K):
        x = jnp.concatenate(
            [in_ref[:, g * SB + 128 * c:g * SB + 128 * (c + 1)] for c in range(4)],
            axis=0,
        )
        out_ref[g * 128:(g + 1) * 128, :] = x.T


@functools.cache
def _make_tc_transpose():
    return pl.pallas_call(
        _tr_body,
        grid=(NBLK,),
        in_specs=[pl.BlockSpec((D, BV), lambda i: (0, i))],
        out_specs=pl.BlockSpec((K * 128, 128), lambda i: (i, 0)),
        out_shape=jax.ShapeDtypeStruct((NBLK * K * 128, 128), jnp.float32),
        compiler_params=pltpu.CompilerParams(
            dimension_semantics=("arbitrary",),
        ),
    )


@functools.cache
def _make_sc_gather():
    mesh = plsc.VectorSubcoreMesh(core_axis_name="c", subcore_axis_name="s")

    @functools.partial(
        pl.kernel,
        out_type=jax.ShapeDtypeStruct((B, NUM_FIELDS * D), jnp.float32),
        mesh=mesh,
        compiler_params=pltpu.CompilerParams(use_tc_tiling_on_sc=False),
        scratch_types=[
            pltpu.VMEM((BPW,), jnp.int32),
            pltpu.VMEM((BPW, D), jnp.float32),
            pltpu.SemaphoreType.DMA,
        ],
    )
    def k(sfT_hbm, tab_hbm, out_hbm, idx_v, rows_v, sem):
        wid = lax.axis_index("s") * NC + lax.axis_index("c")
        b0 = wid * BPW

        def field_body(f, carry):
            pltpu.sync_copy(sfT_hbm.at[f, pl.ds(b0, BPW)], idx_v)

            def remap_body(i, carry2):
                s16 = pl.ds(i * L, L)
                v = idx_v[s16] + f * VOCAB
                idx_v[s16] = (
                    ((v >> 9) << 9) + ((v & 127) << 2) + ((v >> 7) & 3)
                )
                return carry2

            lax.fori_loop(0, BPW // L, remap_body, 0)

            cps = []
            for j in range(NG):
                sl = pl.ds(j * GSIZE, GSIZE)
                cps.append(
                    pltpu.async_copy(tab_hbm.at[idx_v.at[sl]], rows_v.at[sl], sem)
                )
            for cp in cps:
                cp.wait()
            pltpu.sync_copy(rows_v, out_hbm.at[pl.ds(b0, BPW), pl.ds(f * D, D)])
            return carry

        lax.fori_loop(0, NUM_FIELDS, field_body, 0)

    return k


def kernel(sparse_features, tables):
    tabT = tables.T
    lin = _make_tc_transpose()(tabT)
    tab_rm = jnp.reshape(lin, (lin.shape[0] * 4, D))
    sfT = sparse_features.T.astype(jnp.int32)
    out = _make_sc_gather()(sfT, tab_rm)
    return out.reshape(B, NUM_FIELDS, D)


# speedup vs baseline: 2.9100x; 1.0082x over previous
import functools

import jax
import jax.numpy as jnp
from jax import lax
from jax.experimental import pallas as pl
from jax.experimental.pallas import tpu as pltpu
from jax.experimental.pallas import tpu_sc as plsc

NUM_FIELDS = 26
VOCAB = 100000
D = 32
B = 16384
NC, NS, L = 2, 16, 16
NW = NC * NS
BPW = B // NW
GSIZE = 128
NG = BPW // GSIZE

ROWS = NUM_FIELDS * VOCAB
SB = 512
K = 16
BV = K * SB

NF_H = NUM_FIELDS // 2
STEPS_A, OFF_A = 159, 0
STEPS_B, OFF_B = 160, 158
ROW0_B = OFF_B * BV


def _tr_body(in_ref, out_ref):
    for g in range(K):
        x = jnp.concatenate(
            [in_ref[:, g * SB + 128 * c:g * SB + 128 * (c + 1)] for c in range(4)],
            axis=0,
        )
        out_ref[g * 128:(g + 1) * 128, :] = x.T


@functools.lru_cache(maxsize=None)
def _make_tc_transpose(nsteps, off):
    return pl.pallas_call(
        _tr_body,
        grid=(nsteps,),
        in_specs=[pl.BlockSpec((D, BV), lambda i: (0, i + off))],
        out_specs=pl.BlockSpec((K * 128, 128), lambda i: (i, 0)),
        out_shape=jax.ShapeDtypeStruct((nsteps * K * 128, 128), jnp.float32),
        compiler_params=pltpu.CompilerParams(
            dimension_semantics=("arbitrary",),
        ),
    )


@functools.lru_cache(maxsize=None)
def _make_sc_gather(f0, row0):
    mesh = plsc.VectorSubcoreMesh(core_axis_name="c", subcore_axis_name="s")

    @functools.partial(
        pl.kernel,
        out_type=jax.ShapeDtypeStruct((B, NF_H * D), jnp.float32),
        mesh=mesh,
        compiler_params=pltpu.CompilerParams(use_tc_tiling_on_sc=False),
        scratch_types=[
            pltpu.VMEM((BPW,), jnp.int32),
            pltpu.VMEM((BPW, D), jnp.float32),
            pltpu.SemaphoreType.DMA,
        ],
    )
    def k(sfT_hbm, tab_hbm, out_hbm, idx_v, rows_v, sem):
        wid = lax.axis_index("s") * NC + lax.axis_index("c")
        b0 = wid * BPW

        def field_body(f, carry):
            pltpu.sync_copy(sfT_hbm.at[f0 + f, pl.ds(b0, BPW)], idx_v)
            base = f * VOCAB + (f0 * VOCAB - row0)

            def remap_body(i, carry2):
                s16 = pl.ds(i * L, L)
                v = idx_v[s16] + base
                idx_v[s16] = (
                    ((v >> 9) << 9) + ((v & 127) << 2) + ((v >> 7) & 3)
                )
                return carry2

            lax.fori_loop(0, BPW // L, remap_body, 0)

            cps = []
            for j in range(NG):
                sl = pl.ds(j * GSIZE, GSIZE)
                cps.append(
                    pltpu.async_copy(tab_hbm.at[idx_v.at[sl]], rows_v.at[sl], sem)
                )
            for cp in cps:
                cp.wait()
            pltpu.sync_copy(rows_v, out_hbm.at[pl.ds(b0, BPW), pl.ds(f * D, D)])
            return carry

        lax.fori_loop(0, NF_H, field_body, 0)

    return k


def kernel(sparse_features, tables):
    tabT = tables.T
    sfT = sparse_features.T.astype(jnp.int32)

    linA = _make_tc_transpose(STEPS_A, OFF_A)(tabT)
    tabA = jnp.reshape(linA, (linA.shape[0] * 4, D))
    outA = _make_sc_gather(0, 0)(sfT, tabA)

    linB = _make_tc_transpose(STEPS_B, OFF_B)(tabT)
    tabB = jnp.reshape(linB, (linB.shape[0] * 4, D))
    outB = _make_sc_gather(NF_H, ROW0_B)(sfT, tabB)

    out = jnp.concatenate([outA, outB], axis=1)
    return out.reshape(B, NUM_FIELDS, D)
